# baseline (speedup 1.0000x reference)
import functools

import jax
import jax.numpy as jnp
from jax import lax
from jax.experimental import pallas as pl
from jax.experimental.pallas import tpu as pltpu
from jax.experimental.pallas import tpu_sc as plsc

VOCAB = 100000
D = 128
B = 4096
CTX = 20

NC = 2
NS = 16
NW = NC * NS
LANES = 16

BPW = B // NW
CHUNK = 4
NCHUNK = BPW // CHUNK
IDX_PER_CHUNK = CHUNK * CTX


def _gather_mean(inp_grouped, table):
  mesh = plsc.VectorSubcoreMesh(core_axis_name="c", subcore_axis_name="s")

  @functools.partial(
      pl.kernel,
      out_type=jax.ShapeDtypeStruct((B, D), jnp.float32),
      mesh=mesh,
      scratch_types=[
          pltpu.VMEM((NCHUNK, IDX_PER_CHUNK), jnp.int32),
          pltpu.VMEM((IDX_PER_CHUNK, D), jnp.float32),
          pltpu.VMEM((BPW, D), jnp.float32),
          pltpu.SemaphoreType.DMA,
      ],
  )
  def k(inp_hbm, table_hbm, out_hbm, idx_v, rows_v, avg_v, sem):
    wid = lax.axis_index("s") * NC + lax.axis_index("c")
    base = wid * BPW
    pltpu.sync_copy(inp_hbm.at[wid], idx_v)

    def chunk_body(c, carry):
      pltpu.async_copy(table_hbm.at[idx_v.at[c]], rows_v, sem).wait()
      for j in range(CHUNK):
        row0 = j * CTX
        for d in range(D // LANES):
          sl = pl.ds(d * LANES, LANES)

          def red(t, acc):
            return acc + rows_v[row0 + t, sl]

          acc = lax.fori_loop(1, CTX, red, rows_v[row0, sl])
          avg_v[c * CHUNK + j, sl] = acc * (1.0 / CTX)
      return carry

    lax.fori_loop(0, NCHUNK, chunk_body, 0)
    pltpu.sync_copy(avg_v, out_hbm.at[pl.ds(base, BPW)])

  return k(inp_grouped, table)


BM = 2048
BN = 512


def _mm_body(x_ref, w_ref, b_ref, o_ref):
  o_ref[...] = lax.dot_general(
      x_ref[...], w_ref[...],
      dimension_numbers=(((1,), (1,)), ((), ())),
      preferred_element_type=jnp.float32) + b_ref[...]


def _project(avg, W, b2):
  nm = B // BM
  nn = pl.cdiv(VOCAB, BN)
  return pl.pallas_call(
      _mm_body,
      grid=(nm, nn),
      in_specs=[
          pl.BlockSpec((BM, D), lambda m, n: (m, 0)),
          pl.BlockSpec((BN, D), lambda m, n: (n, 0)),
          pl.BlockSpec((1, BN), lambda m, n: (0, n)),
      ],
      out_specs=pl.BlockSpec((BM, BN), lambda m, n: (m, n)),
      out_shape=jax.ShapeDtypeStruct((B, VOCAB), jnp.float32),
      compiler_params=pltpu.CompilerParams(
          dimension_semantics=("parallel", "arbitrary")),
  )(avg, W, b2)


def kernel(inp, embed_table, W, b):
  inp_grouped = inp.astype(jnp.int32).reshape(NW, NCHUNK, IDX_PER_CHUNK)
  avg = _gather_mean(inp_grouped, embed_table)
  return _project(avg, W, b.reshape(1, VOCAB))
